# initial kernel scaffold (unmeasured)
import jax
import jax.numpy as jnp
from jax import lax
from jax.experimental import pallas as pl
from jax.experimental.pallas import tpu as pltpu

N_DEV = 4
SQ = 1024
SKV0 = 1024
W1 = 128
HQ = 32
H_PER = 8
DH = 128
WIN = 128
SCALE = 0.08838834764831843
F32 = jnp.float32
BF16 = jnp.bfloat16


def kernel(x, Wq, K_ext, V_ext, Wo):
    def body(x_ref, wq_ref, k_ref, v_ref, wo_ref, out_ref,
             kT, vT, k1T, v1T, kwin0, vwin0, kwin1, vwin1,
             ctx_ref, pstage, precv,
             sendA, recvA0, recvA1, sendC, recvC):
        my = lax.axis_index("i")

        def rdma_A0(p, tensor):
            src, dst, slot = ((kT, kwin0, 0) if tensor == "k"
                              else (vT, vwin0, 1))
            return pltpu.make_async_remote_copy(
                src_ref=src.at[pl.ds(8 * p, H_PER)],
                dst_ref=dst,
                send_sem=sendA.at[p, slot],
                recv_sem=recvA0.at[slot],
                device_id=(p,),
                device_id_type=pl.DeviceIdType.MESH,
            )

        def rdma_A1(p, tensor):
            src, dst, slot = ((k1T, kwin1, 0) if tensor == "k"
                              else (v1T, vwin1, 1))
            return pltpu.make_async_remote_copy(
                src_ref=src.at[pl.ds(8 * p, H_PER)],
                dst_ref=dst,
                send_sem=sendA.at[p, slot],
                recv_sem=recvA1.at[slot],
                device_id=(p,),
                device_id_type=pl.DeviceIdType.MESH,
            )

        def rdma_C(s, p):
            return pltpu.make_async_remote_copy(
                src_ref=pstage,
                dst_ref=precv.at[s],
                send_sem=sendC.at[p],
                recv_sem=recvC.at[s],
                device_id=(p,),
                device_id_type=pl.DeviceIdType.MESH,
            )

        @pl.when(my == 0)
        def _():
            for h in range(HQ):
                kT[h, :, :] = k_ref[0, :, h, :].astype(BF16)
                vT[h, :, :] = v_ref[0, :, h, :].astype(BF16)
            kwin0[...] = kT[pl.ds(0, H_PER), :, :]
            vwin0[...] = vT[pl.ds(0, H_PER), :, :]
            for p in (1, 2, 3):
                rdma_A0(p, "k").start()
                rdma_A0(p, "v").start()

        @pl.when(my == 1)
        def _():
            for h in range(HQ):
                k1T[h, :, :] = k_ref[0, pl.ds(0, W1), h, :].astype(BF16)
                v1T[h, :, :] = v_ref[0, pl.ds(0, W1), h, :].astype(BF16)
            kwin1[...] = k1T[pl.ds(H_PER, H_PER), :, :]
            vwin1[...] = v1T[pl.ds(H_PER, H_PER), :, :]
            for p in (0, 2, 3):
                rdma_A1(p, "k").start()
                rdma_A1(p, "v").start()

        xb = x_ref[0, :, :].astype(BF16)
        wqb = wq_ref[...].astype(BF16)
        q = jnp.dot(xb, wqb, preferred_element_type=F32) * SCALE
        qb = q.astype(BF16)

        qi0 = lax.broadcasted_iota(jnp.int32, (SQ, SKV0), 0)
        kj0 = lax.broadcasted_iota(jnp.int32, (SQ, SKV0), 1)
        mask0 = jnp.abs(qi0 - kj0) <= WIN
        qi1 = lax.broadcasted_iota(jnp.int32, (SQ, W1), 0)
        kj1 = lax.broadcasted_iota(jnp.int32, (SQ, W1), 1) + SKV0
        mask1 = jnp.abs(qi1 - kj1) <= WIN

        @pl.when(my != 0)
        def _():
            rdma_A0(0, "k").wait_recv()
            rdma_A0(0, "v").wait_recv()

        @pl.when(my != 1)
        def _():
            rdma_A1(1, "k").wait_recv()
            rdma_A1(1, "v").wait_recv()

        for h in range(H_PER):
            qh = qb[:, pl.ds(h * DH, DH)]
            s0 = lax.dot_general(qh, kwin0[h, :, :],
                                 (((1,), (1,)), ((), ())),
                                 preferred_element_type=F32)
            s1 = lax.dot_general(qh, kwin1[h, :, :],
                                 (((1,), (1,)), ((), ())),
                                 preferred_element_type=F32)
            s0 = jnp.where(mask0, s0, -1e9)
            s1 = jnp.where(mask1, s1, -1e9)
            m = jnp.maximum(jnp.max(s0, axis=1, keepdims=True),
                            jnp.max(s1, axis=1, keepdims=True))
            e0 = jnp.exp(s0 - m)
            e1 = jnp.exp(s1 - m)
            den = (jnp.sum(e0, axis=1, keepdims=True)
                   + jnp.sum(e1, axis=1, keepdims=True))
            c = (jnp.dot(e0.astype(BF16), vwin0[h, :, :],
                         preferred_element_type=F32)
                 + jnp.dot(e1.astype(BF16), vwin1[h, :, :],
                           preferred_element_type=F32))
            ctx_ref[:, pl.ds(h * DH, DH)] = (c / den).astype(BF16)

        wob = wo_ref[...].astype(BF16)
        partial = jnp.dot(ctx_ref[...], wob, preferred_element_type=F32)
        pstage[...] = partial.astype(BF16)

        for s in range(N_DEV):
            @pl.when(my == s)
            def _(s=s):
                for p in range(N_DEV):
                    if p != s:
                        rdma_C(s, p).start()

        for s in range(N_DEV):
            @pl.when(my != s)
            def _(s=s):
                rdma_C(s, 0).wait_recv()

        total = partial
        for p in range(N_DEV):
            contrib = precv[p, :, :].astype(F32)
            total = total + jnp.where(my == p, 0.0, contrib)
        out_ref[0, :, :] = total

        @pl.when(my == 0)
        def _():
            for p in (1, 2, 3):
                rdma_A0(p, "k").wait_send()
                rdma_A0(p, "v").wait_send()

        @pl.when(my == 1)
        def _():
            for p in (0, 2, 3):
                rdma_A1(p, "k").wait_send()
                rdma_A1(p, "v").wait_send()

        for s in range(N_DEV):
            @pl.when(my == s)
            def _(s=s):
                for p in range(N_DEV):
                    if p != s:
                        rdma_C(s, p).wait_send()

    return pl.pallas_call(
        body,
        out_shape=jax.ShapeDtypeStruct((1, SQ, SQ), F32),
        in_specs=[pl.BlockSpec(memory_space=pltpu.VMEM)] * 5,
        out_specs=pl.BlockSpec(memory_space=pltpu.VMEM),
        scratch_shapes=[
            pltpu.VMEM((HQ, SKV0, DH), BF16),
            pltpu.VMEM((HQ, SKV0, DH), BF16),
            pltpu.VMEM((HQ, W1, DH), BF16),
            pltpu.VMEM((HQ, W1, DH), BF16),
            pltpu.VMEM((H_PER, SKV0, DH), BF16),
            pltpu.VMEM((H_PER, SKV0, DH), BF16),
            pltpu.VMEM((H_PER, W1, DH), BF16),
            pltpu.VMEM((H_PER, W1, DH), BF16),
            pltpu.VMEM((SQ, H_PER * DH), BF16),
            pltpu.VMEM((SQ, SQ), BF16),
            pltpu.VMEM((N_DEV, SQ, SQ), BF16),
            pltpu.SemaphoreType.DMA((N_DEV, 2)),
            pltpu.SemaphoreType.DMA((2,)),
            pltpu.SemaphoreType.DMA((2,)),
            pltpu.SemaphoreType.DMA((N_DEV,)),
            pltpu.SemaphoreType.DMA((N_DEV,)),
        ],
        compiler_params=pltpu.CompilerParams(collective_id=0),
    )(x, Wq, K_ext, V_ext, Wo)


# baseline (device time: 191538 ns/iter reference)
import jax
import jax.numpy as jnp
from jax import lax
from jax.experimental import pallas as pl
from jax.experimental.pallas import tpu as pltpu

N_DEV = 4
SQ = 1024
SKV0 = 1024
W1 = 128
HQ = 32
H_PER = 8
DH = 128
WIN = 128
SCALE = 0.08838834764831843
F32 = jnp.float32
BF16 = jnp.bfloat16


def _slot(s, d):
    return s if s < d else s - 1


def kernel(x, Wq, K_ext, V_ext, Wo):
    wob = Wo.astype(BF16)
    kT = jnp.transpose(K_ext[0].astype(BF16), (1, 0, 2))
    vT = jnp.transpose(V_ext[0].astype(BF16), (1, 0, 2))
    q = jnp.dot(x[0].astype(BF16), Wq.astype(BF16),
                preferred_element_type=F32) * SCALE
    qbi = q.astype(BF16)

    def body(q_ref, kT_ref, vT_ref, wo_ref, out_ref,
             kwin0, vwin0, kwin1, vwin1, ctx_ref, precv,
             sendA, recvA0, recvA1, sendC, recvC, localsem):
        my = lax.axis_index("i")

        def rdma_A0(p, tensor):
            src, dst, slot = ((kT_ref, kwin0, 0) if tensor == "k"
                              else (vT_ref, vwin0, 1))
            return pltpu.make_async_remote_copy(
                src_ref=src.at[pl.ds(H_PER * p, H_PER), :, :],
                dst_ref=dst,
                send_sem=sendA.at[p, slot],
                recv_sem=recvA0.at[slot],
                device_id=(p,),
                device_id_type=pl.DeviceIdType.MESH,
            )

        def rdma_A1(p, tensor):
            src, dst, slot = ((kT_ref, kwin1, 0) if tensor == "k"
                              else (vT_ref, vwin1, 1))
            return pltpu.make_async_remote_copy(
                src_ref=src.at[pl.ds(H_PER * p, H_PER), pl.ds(0, W1), :],
                dst_ref=dst,
                send_sem=sendA.at[p, slot],
                recv_sem=recvA1.at[slot],
                device_id=(p,),
                device_id_type=pl.DeviceIdType.MESH,
            )

        def rdma_C(s, p):
            return pltpu.make_async_remote_copy(
                src_ref=ctx_ref,
                dst_ref=precv.at[_slot(s, p)],
                send_sem=sendC.at[p],
                recv_sem=recvC.at[s],
                device_id=(p,),
                device_id_type=pl.DeviceIdType.MESH,
            )

        @pl.when(my == 0)
        def _():
            for p in (1, 2, 3):
                rdma_A0(p, "k").start()
                rdma_A0(p, "v").start()
            ck = pltpu.make_async_copy(
                kT_ref.at[pl.ds(0, H_PER), :, :], kwin0, localsem.at[0])
            cv = pltpu.make_async_copy(
                vT_ref.at[pl.ds(0, H_PER), :, :], vwin0, localsem.at[1])
            ck.start()
            cv.start()
            ck.wait()
            cv.wait()

        @pl.when(my == 1)
        def _():
            for p in (0, 2, 3):
                rdma_A1(p, "k").start()
                rdma_A1(p, "v").start()
            ck = pltpu.make_async_copy(
                kT_ref.at[pl.ds(H_PER, H_PER), pl.ds(0, W1), :],
                kwin1, localsem.at[0])
            cv = pltpu.make_async_copy(
                vT_ref.at[pl.ds(H_PER, H_PER), pl.ds(0, W1), :],
                vwin1, localsem.at[1])
            ck.start()
            cv.start()
            ck.wait()
            cv.wait()

        @pl.when(my != 0)
        def _():
            rdma_A0(0, "k").wait_recv()
            rdma_A0(0, "v").wait_recv()

        @pl.when(my != 1)
        def _():
            rdma_A1(1, "k").wait_recv()
            rdma_A1(1, "v").wait_recv()

        QB, KB = 256, 512
        NB = SQ // QB
        qb = q_ref[...]
        for b in range(NB):
            r0 = QB * b
            start = max(0, r0 - WIN)
            qi = lax.broadcasted_iota(jnp.int32, (QB, KB), 0) + r0
            kj = lax.broadcasted_iota(jnp.int32, (QB, KB), 1) + start
            mask = jnp.abs(qi - kj) <= WIN
            for h in range(H_PER):
                qh = qb[r0:r0 + QB, h * DH:(h + 1) * DH]
                if b < NB - 1:
                    s = lax.dot_general(qh, kwin0[h, start:start + KB, :],
                                        (((1,), (1,)), ((), ())),
                                        preferred_element_type=F32)
                else:
                    s = jnp.concatenate([
                        lax.dot_general(qh, kwin0[h, start:SKV0, :],
                                        (((1,), (1,)), ((), ())),
                                        preferred_element_type=F32),
                        lax.dot_general(qh, kwin1[h, :, :],
                                        (((1,), (1,)), ((), ())),
                                        preferred_element_type=F32),
                    ], axis=1)
                s = jnp.where(mask, s, -1e9)
                m = jnp.max(s, axis=1, keepdims=True)
                e = jnp.exp(s - m)
                den = jnp.sum(e, axis=1, keepdims=True)
                eb = e.astype(BF16)
                if b < NB - 1:
                    c = jnp.dot(eb, vwin0[h, start:start + KB, :],
                                preferred_element_type=F32)
                else:
                    n0 = SKV0 - start
                    c = (jnp.dot(eb[:, :n0], vwin0[h, start:SKV0, :],
                                 preferred_element_type=F32)
                         + jnp.dot(eb[:, n0:], vwin1[h, :, :],
                                   preferred_element_type=F32))
                ctx_ref[r0:r0 + QB, h * DH:(h + 1) * DH] = (c / den).astype(BF16)

        partial = jnp.dot(ctx_ref[...], wo_ref[...], preferred_element_type=F32)
        ctx_ref[...] = partial.astype(BF16)

        for s in range(N_DEV):
            @pl.when(my == s)
            def _(s=s):
                for p in range(N_DEV):
                    if p != s:
                        rdma_C(s, p).start()

        for d in range(N_DEV):
            @pl.when(my == d)
            def _(d=d):
                total = partial
                for s in range(N_DEV):
                    if s != d:
                        rdma_C(s, d).wait_recv()
                        total = total + precv[_slot(s, d), :, :].astype(F32)
                out_ref[0, :, :] = total

        @pl.when(my == 0)
        def _():
            for p in (1, 2, 3):
                rdma_A0(p, "k").wait_send()
                rdma_A0(p, "v").wait_send()

        @pl.when(my == 1)
        def _():
            for p in (0, 2, 3):
                rdma_A1(p, "k").wait_send()
                rdma_A1(p, "v").wait_send()

        for s in range(N_DEV):
            @pl.when(my == s)
            def _(s=s):
                for p in range(N_DEV):
                    if p != s:
                        rdma_C(s, p).wait_send()

    return pl.pallas_call(
        body,
        out_shape=jax.ShapeDtypeStruct((1, SQ, SQ), F32),
        in_specs=[
            pl.BlockSpec(memory_space=pltpu.VMEM),
            pl.BlockSpec(memory_space=pltpu.MemorySpace.HBM),
            pl.BlockSpec(memory_space=pltpu.MemorySpace.HBM),
            pl.BlockSpec(memory_space=pltpu.VMEM),
        ],
        out_specs=pl.BlockSpec(memory_space=pltpu.VMEM),
        scratch_shapes=[
            pltpu.VMEM((H_PER, SKV0, DH), BF16),
            pltpu.VMEM((H_PER, SKV0, DH), BF16),
            pltpu.VMEM((H_PER, W1, DH), BF16),
            pltpu.VMEM((H_PER, W1, DH), BF16),
            pltpu.VMEM((SQ, H_PER * DH), BF16),
            pltpu.VMEM((N_DEV - 1, SQ, SQ), BF16),
            pltpu.SemaphoreType.DMA((N_DEV, 2)),
            pltpu.SemaphoreType.DMA((2,)),
            pltpu.SemaphoreType.DMA((2,)),
            pltpu.SemaphoreType.DMA((N_DEV,)),
            pltpu.SemaphoreType.DMA((N_DEV,)),
            pltpu.SemaphoreType.DMA((2,)),
        ],
    )(qbi, kT, vT, wob)


# device time: 149711 ns/iter; 1.2794x vs baseline; 1.2794x over previous
import jax
import jax.numpy as jnp
from jax import lax
from jax.experimental import pallas as pl
from jax.experimental.pallas import tpu as pltpu

N_DEV = 4
SQ = 1024
SKV0 = 1024
W1 = 128
HQ = 32
H_PER = 8
DH = 128
WIN = 128
SCALE = 0.08838834764831843
F32 = jnp.float32
BF16 = jnp.bfloat16


def _slot(s, d):
    return s if s < d else s - 1


def kernel(x, Wq, K_ext, V_ext, Wo):
    wob = Wo.astype(BF16)
    kT = jnp.transpose(K_ext[0].astype(BF16), (1, 0, 2))
    vT = jnp.transpose(V_ext[0].astype(BF16), (1, 0, 2))
    q = jnp.dot(x[0].astype(BF16), Wq.astype(BF16),
                preferred_element_type=F32) * SCALE
    qbi = q.astype(BF16)

    def body(q_ref, kT_ref, vT_ref, wo_ref, out_ref,
             kwin0, vwin0, kwin1, vwin1, kfwd, vfwd, ctx_ref, rsbuf, agbuf,
             sendA, recvA0, recvA1, sendC, recvC, sendC2, recvC2,
             sendF, recvF, localsem):
        my = lax.axis_index("i")
        RB = SQ // N_DEV

        def rdma_A0(p, tensor):
            src, dst, slot = ((kT_ref, kwin0, 0) if tensor == "k"
                              else (vT_ref, vwin0, 1))
            return pltpu.make_async_remote_copy(
                src_ref=src.at[pl.ds(H_PER * p, H_PER), :, :],
                dst_ref=dst,
                send_sem=sendA.at[p, slot],
                recv_sem=recvA0.at[slot],
                device_id=(p,),
                device_id_type=pl.DeviceIdType.MESH,
            )

        def rdma_A1(p, tensor):
            src, dst, slot = ((kT_ref, kwin1, 0) if tensor == "k"
                              else (vT_ref, vwin1, 1))
            return pltpu.make_async_remote_copy(
                src_ref=src.at[pl.ds(H_PER * p, H_PER), pl.ds(0, W1), :],
                dst_ref=dst,
                send_sem=sendA.at[p, slot],
                recv_sem=recvA1.at[slot],
                device_id=(p,),
                device_id_type=pl.DeviceIdType.MESH,
            )

        def rdma_RS(s, p):
            return pltpu.make_async_remote_copy(
                src_ref=ctx_ref.at[pl.ds(RB * p, RB), :],
                dst_ref=rsbuf.at[_slot(s, p)],
                send_sem=sendC.at[p],
                recv_sem=recvC.at[s],
                device_id=(p,),
                device_id_type=pl.DeviceIdType.MESH,
            )

        def rdma_AG(s, p):
            return pltpu.make_async_remote_copy(
                src_ref=ctx_ref.at[pl.ds(RB * s, RB), :],
                dst_ref=agbuf.at[_slot(s, p)],
                send_sem=sendC2.at[p],
                recv_sem=recvC2.at[s],
                device_id=(p,),
                device_id_type=pl.DeviceIdType.MESH,
            )

        def rdma_fwd0(tensor):
            src, dst, slot, relay = ((kT_ref, kfwd, 0, 1) if tensor == "k"
                                     else (vT_ref, vfwd, 1, 3))
            return pltpu.make_async_remote_copy(
                src_ref=src.at[pl.ds(H_PER * 2, H_PER), :, :],
                dst_ref=dst,
                send_sem=sendA.at[2, slot],
                recv_sem=recvF.at[slot],
                device_id=(relay,),
                device_id_type=pl.DeviceIdType.MESH,
            )

        def rdma_relay(tensor):
            src, dst, slot = ((kfwd, kwin0, 0) if tensor == "k"
                              else (vfwd, vwin0, 1))
            return pltpu.make_async_remote_copy(
                src_ref=src,
                dst_ref=dst,
                send_sem=sendF.at[slot],
                recv_sem=recvA0.at[slot],
                device_id=(2,),
                device_id_type=pl.DeviceIdType.MESH,
            )

        @pl.when(my == 0)
        def _():
            rdma_fwd0("k").start()
            rdma_fwd0("v").start()
            for p in (1, 3):
                rdma_A0(p, "k").start()
                rdma_A0(p, "v").start()
            ck = pltpu.make_async_copy(
                kT_ref.at[pl.ds(0, H_PER), :, :], kwin0, localsem.at[0])
            cv = pltpu.make_async_copy(
                vT_ref.at[pl.ds(0, H_PER), :, :], vwin0, localsem.at[1])
            ck.start()
            cv.start()
            ck.wait()
            cv.wait()

        @pl.when(my == 1)
        def _():
            for p in (0, 2, 3):
                rdma_A1(p, "k").start()
                rdma_A1(p, "v").start()
            ck = pltpu.make_async_copy(
                kT_ref.at[pl.ds(H_PER, H_PER), pl.ds(0, W1), :],
                kwin1, localsem.at[0])
            cv = pltpu.make_async_copy(
                vT_ref.at[pl.ds(H_PER, H_PER), pl.ds(0, W1), :],
                vwin1, localsem.at[1])
            ck.start()
            cv.start()
            ck.wait()
            cv.wait()

        @pl.when(my == 1)
        def _():
            rdma_fwd0("k").wait_recv()
            rdma_relay("k").start()

        @pl.when(my == 3)
        def _():
            rdma_fwd0("v").wait_recv()
            rdma_relay("v").start()

        @pl.when(my != 0)
        def _():
            rdma_A0(0, "k").wait_recv()
            rdma_A0(0, "v").wait_recv()

        @pl.when(my != 1)
        def _():
            rdma_A1(1, "k").wait_recv()
            rdma_A1(1, "v").wait_recv()

        QB, KB = 256, 512
        NB = SQ // QB
        qb = q_ref[...]
        for b in range(NB):
            r0 = QB * b
            start = max(0, r0 - WIN)
            qi = lax.broadcasted_iota(jnp.int32, (QB, KB), 0) + r0
            kj = lax.broadcasted_iota(jnp.int32, (QB, KB), 1) + start
            mask = jnp.abs(qi - kj) <= WIN
            for h in range(H_PER):
                qh = qb[r0:r0 + QB, h * DH:(h + 1) * DH]
                if b < NB - 1:
                    s = lax.dot_general(qh, kwin0[h, start:start + KB, :],
                                        (((1,), (1,)), ((), ())),
                                        preferred_element_type=F32)
                else:
                    s = jnp.concatenate([
                        lax.dot_general(qh, kwin0[h, start:SKV0, :],
                                        (((1,), (1,)), ((), ())),
                                        preferred_element_type=F32),
                        lax.dot_general(qh, kwin1[h, :, :],
                                        (((1,), (1,)), ((), ())),
                                        preferred_element_type=F32),
                    ], axis=1)
                s = jnp.where(mask, s, -1e9)
                m = jnp.max(s, axis=1, keepdims=True)
                e = jnp.exp(s - m)
                den = jnp.sum(e, axis=1, keepdims=True)
                eb = e.astype(BF16)
                if b < NB - 1:
                    c = jnp.dot(eb, vwin0[h, start:start + KB, :],
                                preferred_element_type=F32)
                else:
                    n0 = SKV0 - start
                    c = (jnp.dot(eb[:, :n0], vwin0[h, start:SKV0, :],
                                 preferred_element_type=F32)
                         + jnp.dot(eb[:, n0:], vwin1[h, :, :],
                                   preferred_element_type=F32))
                ctx_ref[r0:r0 + QB, h * DH:(h + 1) * DH] = (c / den).astype(BF16)

        partial = jnp.dot(ctx_ref[...], wo_ref[...], preferred_element_type=F32)
        ctx_ref[...] = partial.astype(BF16)

        for s in range(N_DEV):
            @pl.when(my == s)
            def _(s=s):
                for p in range(N_DEV):
                    if p != s:
                        rdma_RS(s, p).start()

        for d in range(N_DEV):
            @pl.when(my == d)
            def _(d=d):
                red = partial[RB * d:RB * (d + 1), :]
                for s in range(N_DEV):
                    if s != d:
                        rdma_RS(s, d).wait_recv()
                        red = red + rsbuf[_slot(s, d), :, :].astype(F32)
                out_ref[0, RB * d:RB * (d + 1), :] = red
                ctx_ref[RB * d:RB * (d + 1), :] = red.astype(BF16)
                for p in range(N_DEV):
                    if p != d:
                        rdma_AG(d, p).start()

        for d in range(N_DEV):
            @pl.when(my == d)
            def _(d=d):
                for s in range(N_DEV):
                    if s != d:
                        rdma_AG(s, d).wait_recv()
                        out_ref[0, RB * s:RB * (s + 1), :] = (
                            agbuf[_slot(s, d), :, :].astype(F32))

        @pl.when(my == 0)
        def _():
            for p in (1, 3):
                rdma_A0(p, "k").wait_send()
                rdma_A0(p, "v").wait_send()
            rdma_fwd0("k").wait_send()
            rdma_fwd0("v").wait_send()

        @pl.when(my == 1)
        def _():
            rdma_relay("k").wait_send()

        @pl.when(my == 3)
        def _():
            rdma_relay("v").wait_send()

        @pl.when(my == 1)
        def _():
            for p in (0, 2, 3):
                rdma_A1(p, "k").wait_send()
                rdma_A1(p, "v").wait_send()

        for s in range(N_DEV):
            @pl.when(my == s)
            def _(s=s):
                for p in range(N_DEV):
                    if p != s:
                        rdma_RS(s, p).wait_send()
                        rdma_AG(s, p).wait_send()

    return pl.pallas_call(
        body,
        out_shape=jax.ShapeDtypeStruct((1, SQ, SQ), F32),
        in_specs=[
            pl.BlockSpec(memory_space=pltpu.VMEM),
            pl.BlockSpec(memory_space=pltpu.MemorySpace.HBM),
            pl.BlockSpec(memory_space=pltpu.MemorySpace.HBM),
            pl.BlockSpec(memory_space=pltpu.VMEM),
        ],
        out_specs=pl.BlockSpec(memory_space=pltpu.VMEM),
        scratch_shapes=[
            pltpu.VMEM((H_PER, SKV0, DH), BF16),
            pltpu.VMEM((H_PER, SKV0, DH), BF16),
            pltpu.VMEM((H_PER, W1, DH), BF16),
            pltpu.VMEM((H_PER, W1, DH), BF16),
            pltpu.VMEM((H_PER, SKV0, DH), BF16),
            pltpu.VMEM((H_PER, SKV0, DH), BF16),
            pltpu.VMEM((SQ, H_PER * DH), BF16),
            pltpu.VMEM((N_DEV - 1, SQ // N_DEV, SQ), BF16),
            pltpu.VMEM((N_DEV - 1, SQ // N_DEV, SQ), BF16),
            pltpu.SemaphoreType.DMA((N_DEV, 2)),
            pltpu.SemaphoreType.DMA((2,)),
            pltpu.SemaphoreType.DMA((2,)),
            pltpu.SemaphoreType.DMA((N_DEV,)),
            pltpu.SemaphoreType.DMA((N_DEV,)),
            pltpu.SemaphoreType.DMA((N_DEV,)),
            pltpu.SemaphoreType.DMA((N_DEV,)),
            pltpu.SemaphoreType.DMA((2,)),
            pltpu.SemaphoreType.DMA((2,)),
            pltpu.SemaphoreType.DMA((2,)),
        ],
    )(qbi, kT, vT, wob)


# device time: 145970 ns/iter; 1.3122x vs baseline; 1.0256x over previous
import jax
import jax.numpy as jnp
from jax import lax
from jax.experimental import pallas as pl
from jax.experimental.pallas import tpu as pltpu

N_DEV = 4
SQ = 1024
SKV0 = 1024
W1 = 128
HQ = 32
H_PER = 8
DH = 128
WIN = 128
SCALE = 0.08838834764831843
F32 = jnp.float32
BF16 = jnp.bfloat16


def _slot(s, d):
    return s if s < d else s - 1


def kernel(x, Wq, K_ext, V_ext, Wo):
    wob = Wo.astype(BF16)
    kT = jnp.transpose(K_ext[0].astype(BF16), (1, 0, 2))
    vT = jnp.transpose(V_ext[0].astype(BF16), (1, 0, 2))
    q = jnp.dot(x[0].astype(BF16), Wq.astype(BF16),
                preferred_element_type=F32) * SCALE
    qbi = q.astype(BF16)

    def body(q_ref, kT_ref, vT_ref, wo_ref, out_ref,
             kwin0, vwin0, kwin1, vwin1, kfwd, vfwd, ctx_ref, rsbuf, agbuf,
             sendA, recvA0, recvA1, sendC, recvC, sendC2, recvC2,
             sendF, recvF, localsem):
        my = lax.axis_index("i")
        RB = SQ // N_DEV

        def rdma_A0(p, tensor):
            src, dst, slot = ((kT_ref, kwin0, 0) if tensor == "k"
                              else (vT_ref, vwin0, 1))
            return pltpu.make_async_remote_copy(
                src_ref=src.at[pl.ds(H_PER * p, H_PER), :, :],
                dst_ref=dst,
                send_sem=sendA.at[p, slot],
                recv_sem=recvA0.at[slot],
                device_id=(p,),
                device_id_type=pl.DeviceIdType.MESH,
            )

        def rdma_A1(p, tensor):
            src, dst, slot = ((kT_ref, kwin1, 0) if tensor == "k"
                              else (vT_ref, vwin1, 1))
            return pltpu.make_async_remote_copy(
                src_ref=src.at[pl.ds(H_PER * p, H_PER), pl.ds(0, W1), :],
                dst_ref=dst,
                send_sem=sendA.at[p, slot],
                recv_sem=recvA1.at[slot],
                device_id=(p,),
                device_id_type=pl.DeviceIdType.MESH,
            )

        def rdma_RS(s, p):
            return pltpu.make_async_remote_copy(
                src_ref=ctx_ref.at[pl.ds(RB * p, RB), :],
                dst_ref=rsbuf.at[_slot(s, p)],
                send_sem=sendC.at[p],
                recv_sem=recvC.at[s],
                device_id=(p,),
                device_id_type=pl.DeviceIdType.MESH,
            )

        def rdma_AG(s, p):
            return pltpu.make_async_remote_copy(
                src_ref=ctx_ref.at[pl.ds(RB * s, RB), :],
                dst_ref=agbuf.at[_slot(s, p)],
                send_sem=sendC2.at[p],
                recv_sem=recvC2.at[s],
                device_id=(p,),
                device_id_type=pl.DeviceIdType.MESH,
            )

        def rdma_fwd0(tensor):
            src, dst, slot, relay = ((kT_ref, kfwd, 0, 1) if tensor == "k"
                                     else (vT_ref, vfwd, 1, 3))
            return pltpu.make_async_remote_copy(
                src_ref=src.at[pl.ds(H_PER * 2, H_PER), :, :],
                dst_ref=dst,
                send_sem=sendA.at[2, slot],
                recv_sem=recvF.at[slot],
                device_id=(relay,),
                device_id_type=pl.DeviceIdType.MESH,
            )

        def rdma_relay(tensor):
            src, dst, slot = ((kfwd, kwin0, 0) if tensor == "k"
                              else (vfwd, vwin0, 1))
            return pltpu.make_async_remote_copy(
                src_ref=src,
                dst_ref=dst,
                send_sem=sendF.at[slot],
                recv_sem=recvA0.at[slot],
                device_id=(2,),
                device_id_type=pl.DeviceIdType.MESH,
            )

        @pl.when(my == 0)
        def _():
            rdma_fwd0("k").start()
            rdma_fwd0("v").start()
            for p in (1, 3):
                rdma_A0(p, "k").start()
                rdma_A0(p, "v").start()
            ck = pltpu.make_async_copy(
                kT_ref.at[pl.ds(0, H_PER), :, :], kwin0, localsem.at[0])
            cv = pltpu.make_async_copy(
                vT_ref.at[pl.ds(0, H_PER), :, :], vwin0, localsem.at[1])
            ck.start()
            cv.start()
            ck.wait()
            cv.wait()

        @pl.when(my == 1)
        def _():
            for p in (0, 2, 3):
                rdma_A1(p, "k").start()
                rdma_A1(p, "v").start()
            ck = pltpu.make_async_copy(
                kT_ref.at[pl.ds(H_PER, H_PER), pl.ds(0, W1), :],
                kwin1, localsem.at[0])
            cv = pltpu.make_async_copy(
                vT_ref.at[pl.ds(H_PER, H_PER), pl.ds(0, W1), :],
                vwin1, localsem.at[1])
            ck.start()
            cv.start()
            ck.wait()
            cv.wait()

        @pl.when(my == 1)
        def _():
            rdma_fwd0("k").wait_recv()
            rdma_relay("k").start()

        @pl.when(my == 3)
        def _():
            rdma_fwd0("v").wait_recv()
            rdma_relay("v").start()

        @pl.when(my != 0)
        def _():
            rdma_A0(0, "k").wait_recv()
            rdma_A0(0, "v").wait_recv()

        QB, KB = 256, 512
        NB = SQ // QB
        partials = []
        qb = q_ref[...]
        for b in range(NB):
            if b == NB - 1:
                @pl.when(my != 1)
                def _():
                    rdma_A1(1, "k").wait_recv()
                    rdma_A1(1, "v").wait_recv()
            r0 = QB * b
            start = max(0, r0 - WIN)
            qi = lax.broadcasted_iota(jnp.int32, (QB, KB), 0) + r0
            kj = lax.broadcasted_iota(jnp.int32, (QB, KB), 1) + start
            mask = jnp.abs(qi - kj) <= WIN
            for h in range(H_PER):
                qh = qb[r0:r0 + QB, h * DH:(h + 1) * DH]
                if b < NB - 1:
                    s = lax.dot_general(qh, kwin0[h, start:start + KB, :],
                                        (((1,), (1,)), ((), ())),
                                        preferred_element_type=F32)
                else:
                    s = jnp.concatenate([
                        lax.dot_general(qh, kwin0[h, start:SKV0, :],
                                        (((1,), (1,)), ((), ())),
                                        preferred_element_type=F32),
                        lax.dot_general(qh, kwin1[h, :, :],
                                        (((1,), (1,)), ((), ())),
                                        preferred_element_type=F32),
                    ], axis=1)
                s = jnp.where(mask, s, -1e9)
                m = jnp.max(s, axis=1, keepdims=True)
                e = jnp.exp(s - m)
                den = jnp.sum(e, axis=1, keepdims=True)
                eb = e.astype(BF16)
                if b < NB - 1:
                    c = jnp.dot(eb, vwin0[h, start:start + KB, :],
                                preferred_element_type=F32)
                else:
                    n0 = SKV0 - start
                    c = (jnp.dot(eb[:, :n0], vwin0[h, start:SKV0, :],
                                 preferred_element_type=F32)
                         + jnp.dot(eb[:, n0:], vwin1[h, :, :],
                                   preferred_element_type=F32))
                ctx_ref[r0:r0 + QB, h * DH:(h + 1) * DH] = (c / den).astype(BF16)

            pb = jnp.dot(ctx_ref[r0:r0 + QB, :], wo_ref[...],
                         preferred_element_type=F32)
            partials.append(pb)
            ctx_ref[r0:r0 + QB, :] = pb.astype(BF16)
            for s in range(N_DEV):
                if s != b:
                    @pl.when(my == s)
                    def _(s=s, b=b):
                        rdma_RS(s, b).start()

        for d in range(N_DEV):
            @pl.when(my == d)
            def _(d=d):
                red = partials[d]
                for s in range(N_DEV):
                    if s != d:
                        rdma_RS(s, d).wait_recv()
                        red = red + rsbuf[_slot(s, d), :, :].astype(F32)
                out_ref[0, RB * d:RB * (d + 1), :] = red
                ctx_ref[RB * d:RB * (d + 1), :] = red.astype(BF16)
                for p in range(N_DEV):
                    if p != d:
                        rdma_AG(d, p).start()

        for d in range(N_DEV):
            @pl.when(my == d)
            def _(d=d):
                for s in range(N_DEV):
                    if s != d:
                        rdma_AG(s, d).wait_recv()
                        out_ref[0, RB * s:RB * (s + 1), :] = (
                            agbuf[_slot(s, d), :, :].astype(F32))

        @pl.when(my == 0)
        def _():
            for p in (1, 3):
                rdma_A0(p, "k").wait_send()
                rdma_A0(p, "v").wait_send()
            rdma_fwd0("k").wait_send()
            rdma_fwd0("v").wait_send()

        @pl.when(my == 1)
        def _():
            rdma_relay("k").wait_send()

        @pl.when(my == 3)
        def _():
            rdma_relay("v").wait_send()

        @pl.when(my == 1)
        def _():
            for p in (0, 2, 3):
                rdma_A1(p, "k").wait_send()
                rdma_A1(p, "v").wait_send()

        for s in range(N_DEV):
            @pl.when(my == s)
            def _(s=s):
                for p in range(N_DEV):
                    if p != s:
                        rdma_RS(s, p).wait_send()
                        rdma_AG(s, p).wait_send()

    return pl.pallas_call(
        body,
        out_shape=jax.ShapeDtypeStruct((1, SQ, SQ), F32),
        in_specs=[
            pl.BlockSpec(memory_space=pltpu.VMEM),
            pl.BlockSpec(memory_space=pltpu.MemorySpace.HBM),
            pl.BlockSpec(memory_space=pltpu.MemorySpace.HBM),
            pl.BlockSpec(memory_space=pltpu.VMEM),
        ],
        out_specs=pl.BlockSpec(memory_space=pltpu.VMEM),
        scratch_shapes=[
            pltpu.VMEM((H_PER, SKV0, DH), BF16),
            pltpu.VMEM((H_PER, SKV0, DH), BF16),
            pltpu.VMEM((H_PER, W1, DH), BF16),
            pltpu.VMEM((H_PER, W1, DH), BF16),
            pltpu.VMEM((H_PER, SKV0, DH), BF16),
            pltpu.VMEM((H_PER, SKV0, DH), BF16),
            pltpu.VMEM((SQ, H_PER * DH), BF16),
            pltpu.VMEM((N_DEV - 1, SQ // N_DEV, SQ), BF16),
            pltpu.VMEM((N_DEV - 1, SQ // N_DEV, SQ), BF16),
            pltpu.SemaphoreType.DMA((N_DEV, 2)),
            pltpu.SemaphoreType.DMA((2,)),
            pltpu.SemaphoreType.DMA((2,)),
            pltpu.SemaphoreType.DMA((N_DEV,)),
            pltpu.SemaphoreType.DMA((N_DEV,)),
            pltpu.SemaphoreType.DMA((N_DEV,)),
            pltpu.SemaphoreType.DMA((N_DEV,)),
            pltpu.SemaphoreType.DMA((2,)),
            pltpu.SemaphoreType.DMA((2,)),
            pltpu.SemaphoreType.DMA((2,)),
        ],
    )(qbi, kT, vT, wob)
